# Initial kernel scaffold; baseline (speedup 1.0000x reference)
import functools

import jax
import jax.numpy as jnp
from jax import lax
from jax.experimental import pallas as pl
from jax.experimental.pallas import tpu as pltpu
from jax.experimental.pallas import tpu_sc as plsc

VOCAB = 1000000
EMB = 64
B = 16384
NEG = 20

NC = 2
NS = 16
NW = NC * NS
L = 16

PT = B // NW
SB = 64
NCHUNK = PT // SB
NIDX = SB * NEG // 128
SCORES_PER_CHUNK = SB * (NEG + 1)
TOT_SCORES = B * (NEG + 1)


def _sc_scores_kernel(pos_u_h, pos_v_h, neg2_h, uw_h, vw_h, out_h,
                      idx_u, idx_v, idx_n, rows_u, rows_v, rows_n,
                      u_t, scores, sem):
    wid = lax.axis_index("s") * NC + lax.axis_index("c")
    lanes = lax.iota(jnp.int32, L)

    def chunk_body(c, _):
        base_b = wid * PT + c * SB

        pltpu.sync_copy(pos_u_h.at[pl.ds(base_b, SB)], idx_u)
        pltpu.sync_copy(pos_v_h.at[pl.ds(base_b, SB)], idx_v)
        pltpu.sync_copy(neg2_h.at[pl.ds(base_b * NEG // 128, NIDX)], idx_n)

        copies = [
            pltpu.async_copy(uw_h.at[idx_u], rows_u, sem),
            pltpu.async_copy(vw_h.at[idx_v], rows_v, sem),
        ]
        for j in range(NIDX):
            copies.append(
                pltpu.async_copy(vw_h.at[idx_n.at[j]],
                                 rows_n.at[pl.ds(j * 128, 128)], sem))
        for cp in copies:
            cp.wait()

        def group_body(g, _):
            row16 = lanes + g * L
            pos_acc = jnp.zeros((L,), jnp.float32)
            for d in range(EMB):
                col = jnp.full((L,), d, jnp.int32)
                u_d = plsc.load_gather(rows_u, [row16, col])
                v_d = plsc.load_gather(rows_v, [row16, col])
                u_t[d, :] = u_d
                pos_acc = pos_acc + u_d * v_d
            scores[pl.ds(NEG * SB + g * L, L)] = pos_acc

            def neg_body(n, _):
                rown = lanes * NEG + (g * L * NEG + n)
                acc = jnp.zeros((L,), jnp.float32)
                for d in range(EMB):
                    col = jnp.full((L,), d, jnp.int32)
                    n_d = plsc.load_gather(rows_n, [rown, col])
                    acc = acc + n_d * u_t[d, :]
                scores[pl.ds(n * SB + g * L, L)] = -acc
                return 0

            lax.fori_loop(0, NEG, neg_body, 0)
            return 0

        lax.fori_loop(0, SB // L, group_body, 0)

        out_off = (wid * NCHUNK + c) * SCORES_PER_CHUNK
        pltpu.sync_copy(scores, out_h.at[pl.ds(out_off, SCORES_PER_CHUNK)])
        return 0

    lax.fori_loop(0, NCHUNK, chunk_body, 0)


@functools.partial(
    pl.kernel,
    out_type=jax.ShapeDtypeStruct((TOT_SCORES,), jnp.float32),
    mesh=plsc.VectorSubcoreMesh(core_axis_name="c", subcore_axis_name="s"),
    scratch_types=[
        pltpu.VMEM((SB,), jnp.int32),
        pltpu.VMEM((SB,), jnp.int32),
        pltpu.VMEM((NIDX, 128), jnp.int32),
        pltpu.VMEM((SB, EMB), jnp.float32),
        pltpu.VMEM((SB, EMB), jnp.float32),
        pltpu.VMEM((SB * NEG, EMB), jnp.float32),
        pltpu.VMEM((EMB, L), jnp.float32),
        pltpu.VMEM((SCORES_PER_CHUNK,), jnp.float32),
        pltpu.SemaphoreType.DMA,
    ],
)
def _sc_scores(pos_u_h, pos_v_h, neg2_h, uw_h, vw_h, out_h,
               idx_u, idx_v, idx_n, rows_u, rows_v, rows_n,
               u_t, scores, sem):
    _sc_scores_kernel(pos_u_h, pos_v_h, neg2_h, uw_h, vw_h, out_h,
                      idx_u, idx_v, idx_n, rows_u, rows_v, rows_n,
                      u_t, scores, sem)


def _tc_loss_body(x_ref, o_ref):
    x = x_ref[...]
    ls = jnp.minimum(x, 0.0) - jnp.log1p(jnp.exp(-jnp.abs(x)))
    o_ref[0, 0] = -jnp.sum(ls)


def kernel(pos_u, pos_v, neg_v, u_weight, v_weight):
    pos_u = pos_u.astype(jnp.int32)
    pos_v = pos_v.astype(jnp.int32)
    neg2 = neg_v.astype(jnp.int32).reshape(B * NEG // 128, 128)
    scores = _sc_scores(pos_u, pos_v, neg2, u_weight, v_weight)
    scores2d = scores.reshape(TOT_SCORES // 128, 128)
    loss = pl.pallas_call(
        _tc_loss_body,
        out_shape=jax.ShapeDtypeStruct((1, 1), jnp.float32),
    )(scores2d)
    return loss[0, 0]


# speedup vs baseline: 4.2958x; 4.2958x over previous
import functools

import jax
import jax.numpy as jnp
from jax import lax
from jax.experimental import pallas as pl
from jax.experimental.pallas import tpu as pltpu
from jax.experimental.pallas import tpu_sc as plsc

VOCAB = 1000000
EMB = 64
B = 16384
NEG = 20

NC = 2
NS = 16
NW = NC * NS
L = 16
ROWW = 128

PT = B // NW
SB = 32
NCHUNK = PT // SB
NIDX = SB * NEG // 128
TIDX = PT * NEG // 128
SCORES_PER_CHUNK = SB * (NEG + 1)
TOT_SCORES = B * (NEG + 1)

CONV_BN = 2048


def _tc_conv_body(x_ref, o_ref):
    y = x_ref[...].T
    o_ref[...] = jnp.concatenate([y, y], axis=1)


def _tc_relayout(table):
    return pl.pallas_call(
        _tc_conv_body,
        grid=(pl.cdiv(VOCAB, CONV_BN),),
        in_specs=[pl.BlockSpec((EMB, CONV_BN), lambda i: (0, i))],
        out_specs=pl.BlockSpec((CONV_BN, ROWW), lambda i: (i, 0)),
        out_shape=jax.ShapeDtypeStruct((VOCAB, ROWW), jnp.float32),
    )(table.T)


def _sc_scores_kernel(pos_u_h, pos_v_h, neg2_h, uw_h, vw_h, out_h,
                      idx_u, idx_v, idx_n, rows_u, rows_v, rows_n,
                      u_t, scores, sem):
    wid = lax.axis_index("s") * NC + lax.axis_index("c")
    lanes = lax.iota(jnp.int32, L)

    pltpu.sync_copy(neg2_h.at[pl.ds(wid * TIDX, TIDX)], idx_n)
    pltpu.sync_copy(pos_u_h.at[pl.ds(wid * PT, PT)], idx_u)
    pltpu.sync_copy(pos_v_h.at[pl.ds(wid * PT, PT)], idx_v)

    def chunk_body(c, _):
        copies = [
            pltpu.async_copy(uw_h.at[idx_u.at[pl.ds(c * SB, SB)]], rows_u, sem),
            pltpu.async_copy(vw_h.at[idx_v.at[pl.ds(c * SB, SB)]], rows_v, sem),
        ]
        for j in range(NIDX):
            copies.append(
                pltpu.async_copy(vw_h.at[idx_n.at[c * NIDX + j]],
                                 rows_n.at[pl.ds(j * 128, 128)], sem))
        for cp in copies:
            cp.wait()

        def group_body(g, _):
            row16 = lanes + g * L
            pos_acc = jnp.zeros((L,), jnp.float32)
            for d in range(EMB):
                col = jnp.full((L,), d, jnp.int32)
                u_d = plsc.load_gather(rows_u, [row16, col])
                v_d = plsc.load_gather(rows_v, [row16, col])
                u_t[d, :] = u_d
                pos_acc = pos_acc + u_d * v_d
            scores[pl.ds(NEG * SB + g * L, L)] = pos_acc

            def neg_body(n, _):
                rown = lanes * NEG + (g * L * NEG + n)
                acc = jnp.zeros((L,), jnp.float32)
                for d in range(EMB):
                    col = jnp.full((L,), d, jnp.int32)
                    n_d = plsc.load_gather(rows_n, [rown, col])
                    acc = acc + n_d * u_t[d, :]
                scores[pl.ds(n * SB + g * L, L)] = -acc
                return 0

            lax.fori_loop(0, NEG, neg_body, 0)
            return 0

        lax.fori_loop(0, SB // L, group_body, 0)

        out_off = (wid * NCHUNK + c) * SCORES_PER_CHUNK
        pltpu.sync_copy(scores, out_h.at[pl.ds(out_off, SCORES_PER_CHUNK)])
        return 0

    lax.fori_loop(0, NCHUNK, chunk_body, 0)


@functools.partial(
    pl.kernel,
    compiler_params=pltpu.CompilerParams(
        needs_layout_passes=False, use_tc_tiling_on_sc=True),
    out_type=jax.ShapeDtypeStruct((TOT_SCORES,), jnp.float32),
    mesh=plsc.VectorSubcoreMesh(core_axis_name="c", subcore_axis_name="s"),
    scratch_types=[
        pltpu.VMEM((PT,), jnp.int32),
        pltpu.VMEM((PT,), jnp.int32),
        pltpu.VMEM((TIDX, 128), jnp.int32),
        pltpu.VMEM((SB, ROWW), jnp.float32),
        pltpu.VMEM((SB, ROWW), jnp.float32),
        pltpu.VMEM((SB * NEG, ROWW), jnp.float32),
        pltpu.VMEM((EMB, L), jnp.float32),
        pltpu.VMEM((SCORES_PER_CHUNK,), jnp.float32),
        pltpu.SemaphoreType.DMA,
    ],
)
def _sc_scores(pos_u_h, pos_v_h, neg2_h, uw_h, vw_h, out_h,
               idx_u, idx_v, idx_n, rows_u, rows_v, rows_n,
               u_t, scores, sem):
    _sc_scores_kernel(pos_u_h, pos_v_h, neg2_h, uw_h, vw_h, out_h,
                      idx_u, idx_v, idx_n, rows_u, rows_v, rows_n,
                      u_t, scores, sem)


def _tc_loss_body(x_ref, o_ref):
    x = x_ref[...]
    ls = jnp.minimum(x, 0.0) - jnp.log1p(jnp.exp(-jnp.abs(x)))
    o_ref[...] = (-jnp.sum(ls)).reshape(1, 1)


def kernel(pos_u, pos_v, neg_v, u_weight, v_weight):
    pos_u = pos_u.astype(jnp.int32)
    pos_v = pos_v.astype(jnp.int32)
    neg2 = neg_v.astype(jnp.int32).reshape(B * NEG // 128, 128)
    u2 = _tc_relayout(u_weight)
    v2 = _tc_relayout(v_weight)
    scores = _sc_scores(pos_u, pos_v, neg2, u2, v2)
    scores2d = scores.reshape(TOT_SCORES // 128, 128)
    loss = pl.pallas_call(
        _tc_loss_body,
        out_shape=jax.ShapeDtypeStruct((1, 1), jnp.float32),
    )(scores2d)
    return loss[0, 0]


# speedup vs baseline: 5.5936x; 1.3021x over previous
import functools

import jax
import jax.numpy as jnp
from jax import lax
from jax.experimental import pallas as pl
from jax.experimental.pallas import tpu as pltpu
from jax.experimental.pallas import tpu_sc as plsc

VOCAB = 1000000
EMB = 64
B = 16384
NEG = 20

NC = 2
NS = 16
NW = NC * NS
L = 16
ROWW = 128

PT = B // NW
SB = 32
NCHUNK = PT // SB
NIDX = SB * NEG // 128
TIDX = PT * NEG // 128
SCORES_PER_CHUNK = SB * (NEG + 1)
TOT_SCORES = B * (NEG + 1)

CONV_BN = 4096


def _tc_conv_body(x_ref, o_ref):
    x = x_ref[...]
    r = lax.broadcasted_iota(jnp.int32, (EMB, ROWW), 0)
    c = lax.broadcasted_iota(jnp.int32, (EMB, ROWW), 1)
    eye2 = jnp.where(r == (c % EMB), 1.0, 0.0).astype(jnp.float32)
    o_ref[...] = lax.dot_general(
        x, eye2, dimension_numbers=(((0,), (0,)), ((), ())),
        preferred_element_type=jnp.float32)


def _tc_relayout(table):
    return pl.pallas_call(
        _tc_conv_body,
        grid=(pl.cdiv(VOCAB, CONV_BN),),
        in_specs=[pl.BlockSpec((EMB, CONV_BN), lambda i: (0, i))],
        out_specs=pl.BlockSpec((CONV_BN, ROWW), lambda i: (i, 0)),
        out_shape=jax.ShapeDtypeStruct((VOCAB, ROWW), jnp.float32),
    )(table.T)


def _sc_scores_kernel(pos_u_h, pos_v_h, neg2_h, uw_h, vw_h, out_h,
                      idx_u, idx_v, idx_n, rows_u, rows_v, rows_n,
                      u_t, scores, sem):
    wid = lax.axis_index("s") * NC + lax.axis_index("c")
    lanes = lax.iota(jnp.int32, L)

    pltpu.sync_copy(neg2_h.at[pl.ds(wid * TIDX, TIDX)], idx_n)
    pltpu.sync_copy(pos_u_h.at[pl.ds(wid * PT, PT)], idx_u)
    pltpu.sync_copy(pos_v_h.at[pl.ds(wid * PT, PT)], idx_v)

    def chunk_body(c, _):
        copies = [
            pltpu.async_copy(uw_h.at[idx_u.at[pl.ds(c * SB, SB)]], rows_u, sem),
            pltpu.async_copy(vw_h.at[idx_v.at[pl.ds(c * SB, SB)]], rows_v, sem),
        ]
        for j in range(NIDX):
            copies.append(
                pltpu.async_copy(vw_h.at[idx_n.at[c * NIDX + j]],
                                 rows_n.at[pl.ds(j * 128, 128)], sem))
        for cp in copies:
            cp.wait()

        def group_body(g, _):
            row16 = lanes + g * L
            pos_acc = jnp.zeros((L,), jnp.float32)
            for d in range(EMB):
                col = jnp.full((L,), d, jnp.int32)
                u_d = plsc.load_gather(rows_u, [row16, col])
                v_d = plsc.load_gather(rows_v, [row16, col])
                u_t[d, :] = u_d
                pos_acc = pos_acc + u_d * v_d
            scores[pl.ds(NEG * SB + g * L, L)] = pos_acc

            def neg_body(n, _):
                rown = lanes * NEG + (g * L * NEG + n)
                acc = jnp.zeros((L,), jnp.float32)
                for d in range(EMB):
                    col = jnp.full((L,), d, jnp.int32)
                    n_d = plsc.load_gather(rows_n, [rown, col])
                    acc = acc + n_d * u_t[d, :]
                scores[pl.ds(n * SB + g * L, L)] = -acc
                return 0

            lax.fori_loop(0, NEG, neg_body, 0)
            return 0

        lax.fori_loop(0, SB // L, group_body, 0)

        out_off = (wid * NCHUNK + c) * SCORES_PER_CHUNK
        pltpu.sync_copy(scores, out_h.at[pl.ds(out_off, SCORES_PER_CHUNK)])
        return 0

    lax.fori_loop(0, NCHUNK, chunk_body, 0)


@functools.partial(
    pl.kernel,
    compiler_params=pltpu.CompilerParams(
        needs_layout_passes=False, use_tc_tiling_on_sc=True),
    out_type=jax.ShapeDtypeStruct((TOT_SCORES,), jnp.float32),
    mesh=plsc.VectorSubcoreMesh(core_axis_name="c", subcore_axis_name="s"),
    scratch_types=[
        pltpu.VMEM((PT,), jnp.int32),
        pltpu.VMEM((PT,), jnp.int32),
        pltpu.VMEM((TIDX, 128), jnp.int32),
        pltpu.VMEM((SB, ROWW), jnp.float32),
        pltpu.VMEM((SB, ROWW), jnp.float32),
        pltpu.VMEM((SB * NEG, ROWW), jnp.float32),
        pltpu.VMEM((EMB, L), jnp.float32),
        pltpu.VMEM((SCORES_PER_CHUNK,), jnp.float32),
        pltpu.SemaphoreType.DMA,
    ],
)
def _sc_scores(pos_u_h, pos_v_h, neg2_h, uw_h, vw_h, out_h,
               idx_u, idx_v, idx_n, rows_u, rows_v, rows_n,
               u_t, scores, sem):
    _sc_scores_kernel(pos_u_h, pos_v_h, neg2_h, uw_h, vw_h, out_h,
                      idx_u, idx_v, idx_n, rows_u, rows_v, rows_n,
                      u_t, scores, sem)


def _tc_loss_body(x_ref, o_ref):
    x = x_ref[...]
    ls = jnp.minimum(x, 0.0) - jnp.log1p(jnp.exp(-jnp.abs(x)))
    o_ref[...] = (-jnp.sum(ls)).reshape(1, 1)


def kernel(pos_u, pos_v, neg_v, u_weight, v_weight):
    pos_u = pos_u.astype(jnp.int32)
    pos_v = pos_v.astype(jnp.int32)
    neg2 = neg_v.astype(jnp.int32).reshape(B * NEG // 128, 128)
    u2 = _tc_relayout(u_weight)
    v2 = _tc_relayout(v_weight)
    scores = _sc_scores(pos_u, pos_v, neg2, u2, v2)
    scores2d = scores.reshape(TOT_SCORES // 128, 128)
    loss = pl.pallas_call(
        _tc_loss_body,
        out_shape=jax.ShapeDtypeStruct((1, 1), jnp.float32),
    )(scores2d)
    return loss[0, 0]


# speedup vs baseline: 5.9556x; 1.0647x over previous
import functools

import jax
import jax.numpy as jnp
from jax import lax
from jax.experimental import pallas as pl
from jax.experimental.pallas import tpu as pltpu
from jax.experimental.pallas import tpu_sc as plsc

VOCAB = 1000000
EMB = 64
B = 16384
NEG = 20

NC = 2
NS = 16
NW = NC * NS
L = 16
ROWW = 128
KB = EMB // L

PT = B // NW
SB = L
NCHUNK = PT // SB
NIDX = SB * NEG // 64
TIDX = PT * NEG // 64
SCORES_PER_CHUNK = SB * (NEG + 1)
TOT_SCORES = B * (NEG + 1)

CONV_BN = 4096


def _tc_conv_body(x_ref, o_ref):
    x = x_ref[...]
    r = lax.broadcasted_iota(jnp.int32, (EMB, ROWW), 0)
    c = lax.broadcasted_iota(jnp.int32, (EMB, ROWW), 1)
    eye2 = jnp.where(r == (c % EMB), 1.0, 0.0).astype(jnp.float32)
    o_ref[...] = lax.dot_general(
        x, eye2, dimension_numbers=(((0,), (0,)), ((), ())),
        preferred_element_type=jnp.float32)


def _tc_relayout(table):
    return pl.pallas_call(
        _tc_conv_body,
        grid=(pl.cdiv(VOCAB, CONV_BN),),
        in_specs=[pl.BlockSpec((EMB, CONV_BN), lambda i: (0, i))],
        out_specs=pl.BlockSpec((CONV_BN, ROWW), lambda i: (i, 0)),
        out_shape=jax.ShapeDtypeStruct((VOCAB, ROWW), jnp.float32),
    )(table.T)


def _sc_scores_kernel(pos_u_h, pos_v_h, neg2_h, uw_h, vw_h, out_h,
                      idx_u, idx_v, idx_n,
                      bu0, bv0, bn0, bu1, bv1, bn1,
                      accb, scores, sem0, sem1):
    wid = lax.axis_index("s") * NC + lax.axis_index("c")
    lanes = lax.iota(jnp.int32, L)

    pltpu.sync_copy(neg2_h.at[pl.ds(wid * TIDX, TIDX)], idx_n)
    pltpu.sync_copy(pos_u_h.at[pl.ds(wid * PT, PT)], idx_u)
    pltpu.sync_copy(pos_v_h.at[pl.ds(wid * PT, PT)], idx_v)

    def copies(c, bu, bv, bn, sem):
        cps = [
            pltpu.make_async_copy(uw_h.at[idx_u.at[pl.ds(c * SB, SB)]], bu, sem),
            pltpu.make_async_copy(vw_h.at[idx_v.at[pl.ds(c * SB, SB)]], bv, sem),
        ]
        for j in range(NIDX):
            cps.append(pltpu.make_async_copy(
                vw_h.at[idx_n.at[c * NIDX + j]],
                bn.at[pl.ds(j * 64, 64)], sem))
        return cps

    def fire(c, bu, bv, bn, sem):
        for cp in copies(c, bu, bv, bn, sem):
            cp.start()

    def drain(c, bu, bv, bn, sem):
        for cp in copies(c, bu, bv, bn, sem):
            cp.wait()

    def compute(c, bu, bv, bn):
        pos_acc = jnp.zeros((L,), jnp.float32)
        for k in range(KB):
            us = []
            for i in range(L):
                col = jnp.full((L,), k * L + i, jnp.int32)
                u_i = plsc.load_gather(bu, [lanes, col])
                v_i = plsc.load_gather(bv, [lanes, col])
                pos_acc = pos_acc + u_i * v_i
                us.append(u_i)

            def neg_body(n, _):
                rown = lanes * NEG + n
                acc = accb[n, :] if k else jnp.zeros((L,), jnp.float32)
                for i in range(L):
                    col = jnp.full((L,), k * L + i, jnp.int32)
                    acc = acc + plsc.load_gather(bn, [rown, col]) * us[i]
                accb[n, :] = acc
                return 0

            lax.fori_loop(0, NEG, neg_body, 0)

        def store_body(n, _):
            scores[pl.ds(n * L, L)] = -accb[n, :]
            return 0

        lax.fori_loop(0, NEG, store_body, 0)
        scores[pl.ds(NEG * L, L)] = pos_acc
        out_off = (wid * NCHUNK + c) * SCORES_PER_CHUNK
        pltpu.sync_copy(scores, out_h.at[pl.ds(out_off, SCORES_PER_CHUNK)])

    fire(0, bu0, bv0, bn0, sem0)

    def body(t, _):
        c0 = 2 * t
        fire(c0 + 1, bu1, bv1, bn1, sem1)
        drain(c0, bu0, bv0, bn0, sem0)
        compute(c0, bu0, bv0, bn0)

        @pl.when(t < NCHUNK // 2 - 1)
        def _():
            fire(c0 + 2, bu0, bv0, bn0, sem0)

        drain(c0 + 1, bu1, bv1, bn1, sem1)
        compute(c0 + 1, bu1, bv1, bn1)
        return 0

    lax.fori_loop(0, NCHUNK // 2, body, 0)


@functools.partial(
    pl.kernel,
    compiler_params=pltpu.CompilerParams(
        needs_layout_passes=False, use_tc_tiling_on_sc=True),
    out_type=jax.ShapeDtypeStruct((TOT_SCORES,), jnp.float32),
    mesh=plsc.VectorSubcoreMesh(core_axis_name="c", subcore_axis_name="s"),
    scratch_types=[
        pltpu.VMEM((PT,), jnp.int32),
        pltpu.VMEM((PT,), jnp.int32),
        pltpu.VMEM((TIDX, 64), jnp.int32),
        pltpu.VMEM((SB, ROWW), jnp.float32),
        pltpu.VMEM((SB, ROWW), jnp.float32),
        pltpu.VMEM((SB * NEG, ROWW), jnp.float32),
        pltpu.VMEM((SB, ROWW), jnp.float32),
        pltpu.VMEM((SB, ROWW), jnp.float32),
        pltpu.VMEM((SB * NEG, ROWW), jnp.float32),
        pltpu.VMEM((NEG, L), jnp.float32),
        pltpu.VMEM((SCORES_PER_CHUNK,), jnp.float32),
        pltpu.SemaphoreType.DMA,
        pltpu.SemaphoreType.DMA,
    ],
)
def _sc_scores(pos_u_h, pos_v_h, neg2_h, uw_h, vw_h, out_h,
               idx_u, idx_v, idx_n,
               bu0, bv0, bn0, bu1, bv1, bn1,
               accb, scores, sem0, sem1):
    _sc_scores_kernel(pos_u_h, pos_v_h, neg2_h, uw_h, vw_h, out_h,
                      idx_u, idx_v, idx_n,
                      bu0, bv0, bn0, bu1, bv1, bn1,
                      accb, scores, sem0, sem1)


def _tc_loss_body(x_ref, o_ref):
    x = x_ref[...]
    ls = jnp.minimum(x, 0.0) - jnp.log1p(jnp.exp(-jnp.abs(x)))
    o_ref[...] = (-jnp.sum(ls)).reshape(1, 1)


def kernel(pos_u, pos_v, neg_v, u_weight, v_weight):
    pos_u = pos_u.astype(jnp.int32)
    pos_v = pos_v.astype(jnp.int32)
    neg2 = neg_v.astype(jnp.int32).reshape(B * NEG // 64, 64)
    u2 = _tc_relayout(u_weight)
    v2 = _tc_relayout(v_weight)
    scores = _sc_scores(pos_u, pos_v, neg2, u2, v2)
    scores2d = scores.reshape(TOT_SCORES // 128, 128)
    loss = pl.pallas_call(
        _tc_loss_body,
        out_shape=jax.ShapeDtypeStruct((1, 1), jnp.float32),
    )(scores2d)
    return loss[0, 0]


# speedup vs baseline: 6.9531x; 1.1675x over previous
import functools

import jax
import jax.numpy as jnp
from jax import lax
from jax.experimental import pallas as pl
from jax.experimental.pallas import tpu as pltpu
from jax.experimental.pallas import tpu_sc as plsc

VOCAB = 1000000
EMB = 64
B = 16384
NEG = 20

NC = 2
NS = 16
NW = NC * NS
L = 16
ROWW = 128
KB = EMB // L

PT = B // NW
SB = L
NCHUNK = PT // SB
NIDX = SB * NEG // 64
TIDX = PT * NEG // 64
SCORES_PER_CHUNK = SB * (NEG + 1)
TOT_SCORES = B * (NEG + 1)

CONV_BN = 8192


def _tc_conv_body(x_ref, o_ref):
    x = x_ref[...]
    r = lax.broadcasted_iota(jnp.int32, (EMB, ROWW), 0)
    c = lax.broadcasted_iota(jnp.int32, (EMB, ROWW), 1)
    eye2 = jnp.where(r == (c % EMB), 1.0, 0.0).astype(jnp.float32)
    o_ref[...] = lax.dot_general(
        x, eye2, dimension_numbers=(((0,), (0,)), ((), ())),
        preferred_element_type=jnp.float32)


def _tc_relayout(table):
    return pl.pallas_call(
        _tc_conv_body,
        grid=(pl.cdiv(VOCAB, CONV_BN),),
        in_specs=[pl.BlockSpec((EMB, CONV_BN), lambda i: (0, i))],
        out_specs=pl.BlockSpec((CONV_BN, ROWW), lambda i: (i, 0)),
        out_shape=jax.ShapeDtypeStruct((VOCAB, ROWW), jnp.float32),
    )(table.T)


def _sc_scores_kernel(pos_u_h, pos_v_h, neg2_h, uw_h, vw_h, out_h,
                      idx_u, idx_v, idx_n,
                      bu0, bv0, bn0, bu1, bv1, bn1,
                      accb, scores, sem0, sem1):
    wid = lax.axis_index("s") * NC + lax.axis_index("c")
    lanes = lax.iota(jnp.int32, L)

    pltpu.sync_copy(neg2_h.at[pl.ds(wid * TIDX, TIDX)], idx_n)
    pltpu.sync_copy(pos_u_h.at[pl.ds(wid * PT, PT)], idx_u)
    pltpu.sync_copy(pos_v_h.at[pl.ds(wid * PT, PT)], idx_v)

    def copies(c, bu, bv, bn, sem):
        cps = [
            pltpu.make_async_copy(uw_h.at[idx_u.at[pl.ds(c * SB, SB)]], bu, sem),
            pltpu.make_async_copy(vw_h.at[idx_v.at[pl.ds(c * SB, SB)]], bv, sem),
        ]
        for j in range(NIDX):
            cps.append(pltpu.make_async_copy(
                vw_h.at[idx_n.at[c * NIDX + j]],
                bn.at[pl.ds(j * 64, 64)], sem))
        return cps

    def fire(c, bu, bv, bn, sem):
        for cp in copies(c, bu, bv, bn, sem):
            cp.start()

    def drain(c, bu, bv, bn, sem):
        for cp in copies(c, bu, bv, bn, sem):
            cp.wait()

    def compute(c, bu, bv, bn):
        pos_acc = jnp.zeros((L,), jnp.float32)
        for k in range(KB):
            us = []
            for i in range(L):
                col = jnp.full((L,), k * L + i, jnp.int32)
                u_i = plsc.load_gather(bu, [lanes, col])
                v_i = plsc.load_gather(bv, [lanes, col])
                pos_acc = pos_acc + u_i * v_i
                us.append(u_i)

            def neg_body(n, _):
                rown = lanes * NEG + n
                acc = accb[n, :] if k else jnp.zeros((L,), jnp.float32)
                for i in range(L):
                    col = jnp.full((L,), k * L + i, jnp.int32)
                    acc = acc + plsc.load_gather(bn, [rown, col]) * us[i]
                accb[n, :] = acc
                return 0

            lax.fori_loop(0, NEG, neg_body, 0)

        def store_body(n, _):
            scores[pl.ds(n * L, L)] = -accb[n, :]
            return 0

        lax.fori_loop(0, NEG, store_body, 0)
        scores[pl.ds(NEG * L, L)] = pos_acc
        out_off = (wid * NCHUNK + c) * SCORES_PER_CHUNK
        pltpu.sync_copy(scores, out_h.at[pl.ds(out_off, SCORES_PER_CHUNK)])

    fire(0, bu0, bv0, bn0, sem0)

    def body(t, _):
        c0 = 2 * t
        fire(c0 + 1, bu1, bv1, bn1, sem1)
        drain(c0, bu0, bv0, bn0, sem0)
        compute(c0, bu0, bv0, bn0)

        @pl.when(t < NCHUNK // 2 - 1)
        def _():
            fire(c0 + 2, bu0, bv0, bn0, sem0)

        drain(c0 + 1, bu1, bv1, bn1, sem1)
        compute(c0 + 1, bu1, bv1, bn1)
        return 0

    lax.fori_loop(0, NCHUNK // 2, body, 0)


@functools.partial(
    pl.kernel,
    compiler_params=pltpu.CompilerParams(
        needs_layout_passes=False, use_tc_tiling_on_sc=True),
    out_type=jax.ShapeDtypeStruct((TOT_SCORES,), jnp.float32),
    mesh=plsc.VectorSubcoreMesh(core_axis_name="c", subcore_axis_name="s"),
    scratch_types=[
        pltpu.VMEM((PT,), jnp.int32),
        pltpu.VMEM((PT,), jnp.int32),
        pltpu.VMEM((TIDX, 64), jnp.int32),
        pltpu.VMEM((SB, ROWW), jnp.float32),
        pltpu.VMEM((SB, ROWW), jnp.float32),
        pltpu.VMEM((SB * NEG, ROWW), jnp.float32),
        pltpu.VMEM((SB, ROWW), jnp.float32),
        pltpu.VMEM((SB, ROWW), jnp.float32),
        pltpu.VMEM((SB * NEG, ROWW), jnp.float32),
        pltpu.VMEM((NEG, L), jnp.float32),
        pltpu.VMEM((SCORES_PER_CHUNK,), jnp.float32),
        pltpu.SemaphoreType.DMA,
        pltpu.SemaphoreType.DMA,
    ],
)
def _sc_scores(pos_u_h, pos_v_h, neg2_h, uw_h, vw_h, out_h,
               idx_u, idx_v, idx_n,
               bu0, bv0, bn0, bu1, bv1, bn1,
               accb, scores, sem0, sem1):
    _sc_scores_kernel(pos_u_h, pos_v_h, neg2_h, uw_h, vw_h, out_h,
                      idx_u, idx_v, idx_n,
                      bu0, bv0, bn0, bu1, bv1, bn1,
                      accb, scores, sem0, sem1)


def _tc_loss_body(x_ref, o_ref):
    x = x_ref[...]
    ls = jnp.minimum(x, 0.0) - jnp.log1p(jnp.exp(-jnp.abs(x)))
    o_ref[...] = (-jnp.sum(ls)).reshape(1, 1)


def kernel(pos_u, pos_v, neg_v, u_weight, v_weight):
    pos_u = pos_u.astype(jnp.int32)
    pos_v = pos_v.astype(jnp.int32)
    neg2 = neg_v.astype(jnp.int32).reshape(B * NEG // 64, 64)
    u2 = _tc_relayout(u_weight)
    v2 = _tc_relayout(v_weight)
    scores = _sc_scores(pos_u, pos_v, neg2, u2, v2)
    scores2d = scores.reshape(TOT_SCORES // 128, 128)
    loss = pl.pallas_call(
        _tc_loss_body,
        out_shape=jax.ShapeDtypeStruct((1, 1), jnp.float32),
    )(scores2d)
    return loss[0, 0]


# speedup vs baseline: 6.9583x; 1.0007x over previous
import functools

import jax
import jax.numpy as jnp
from jax import lax
from jax.experimental import pallas as pl
from jax.experimental.pallas import tpu as pltpu
from jax.experimental.pallas import tpu_sc as plsc

VOCAB = 1000000
EMB = 64
B = 16384
NEG = 20

NC = 2
NS = 16
NW = NC * NS
L = 16
ROWW = 128
KB = EMB // L

PT = B // NW
SB = L
NCHUNK = PT // SB
VROWS = SB * (NEG + 1)
TV = NCHUNK * VROWS
SCORES_PER_CHUNK = VROWS
TOT_SCORES = B * (NEG + 1)

CONV_BN = 8192


def _tc_conv_body(x_ref, o_ref):
    x = x_ref[...]
    r = lax.broadcasted_iota(jnp.int32, (EMB, ROWW), 0)
    c = lax.broadcasted_iota(jnp.int32, (EMB, ROWW), 1)
    eye2 = jnp.where(r == (c % EMB), 1.0, 0.0).astype(jnp.float32)
    o_ref[...] = lax.dot_general(
        x, eye2, dimension_numbers=(((0,), (0,)), ((), ())),
        preferred_element_type=jnp.float32)


def _tc_relayout(table):
    return pl.pallas_call(
        _tc_conv_body,
        grid=(pl.cdiv(VOCAB, CONV_BN),),
        in_specs=[pl.BlockSpec((EMB, CONV_BN), lambda i: (0, i))],
        out_specs=pl.BlockSpec((CONV_BN, ROWW), lambda i: (i, 0)),
        out_shape=jax.ShapeDtypeStruct((VOCAB, ROWW), jnp.float32),
    )(table.T)


def _sc_scores_kernel(pos_u_h, vall_h, uw_h, vw_h, out_h,
                      idx_u, idx_v,
                      bu0, bv0, bu1, bv1,
                      accb, scores, sem0, sem1):
    wid = lax.axis_index("s") * NC + lax.axis_index("c")
    lanes = lax.iota(jnp.int32, L)

    pltpu.sync_copy(pos_u_h.at[pl.ds(wid * PT, PT)], idx_u)
    pltpu.sync_copy(vall_h.at[pl.ds(wid * TV, TV)], idx_v)

    def copies(c, bu, bv, sem):
        return [
            pltpu.make_async_copy(uw_h.at[idx_u.at[pl.ds(c * SB, SB)]], bu, sem),
            pltpu.make_async_copy(vw_h.at[idx_v.at[pl.ds(c * VROWS, VROWS)]],
                                  bv, sem),
        ]

    def fire(c, bu, bv, sem):
        for cp in copies(c, bu, bv, sem):
            cp.start()

    def drain(c, bu, bv, sem):
        for cp in copies(c, bu, bv, sem):
            cp.wait()

    def compute(c, bu, bv):
        pos_acc = jnp.zeros((L,), jnp.float32)
        for k in range(KB):
            us = []
            for i in range(L):
                col = jnp.full((L,), k * L + i, jnp.int32)
                u_i = plsc.load_gather(bu, [lanes, col])
                v_i = plsc.load_gather(bv, [lanes, col])
                pos_acc = pos_acc + u_i * v_i
                us.append(u_i)

            def neg_body(n, _):
                rown = lanes * NEG + (n + SB)
                acc = accb[n, :] if k else jnp.zeros((L,), jnp.float32)
                for i in range(L):
                    col = jnp.full((L,), k * L + i, jnp.int32)
                    acc = acc + plsc.load_gather(bv, [rown, col]) * us[i]
                accb[n, :] = acc
                return 0

            lax.fori_loop(0, NEG, neg_body, 0)

        def store_body(n, _):
            scores[pl.ds(n * L, L)] = -accb[n, :]
            return 0

        lax.fori_loop(0, NEG, store_body, 0)
        scores[pl.ds(NEG * L, L)] = pos_acc
        out_off = (wid * NCHUNK + c) * SCORES_PER_CHUNK
        pltpu.sync_copy(scores, out_h.at[pl.ds(out_off, SCORES_PER_CHUNK)])

    fire(0, bu0, bv0, sem0)

    def body(t, _):
        c0 = 2 * t
        fire(c0 + 1, bu1, bv1, sem1)
        drain(c0, bu0, bv0, sem0)
        compute(c0, bu0, bv0)

        @pl.when(t < NCHUNK // 2 - 1)
        def _():
            fire(c0 + 2, bu0, bv0, sem0)

        drain(c0 + 1, bu1, bv1, sem1)
        compute(c0 + 1, bu1, bv1)
        return 0

    lax.fori_loop(0, NCHUNK // 2, body, 0)


@functools.partial(
    pl.kernel,
    compiler_params=pltpu.CompilerParams(
        needs_layout_passes=False, use_tc_tiling_on_sc=True),
    out_type=jax.ShapeDtypeStruct((TOT_SCORES,), jnp.float32),
    mesh=plsc.VectorSubcoreMesh(core_axis_name="c", subcore_axis_name="s"),
    scratch_types=[
        pltpu.VMEM((PT,), jnp.int32),
        pltpu.VMEM((TV,), jnp.int32),
        pltpu.VMEM((SB, ROWW), jnp.float32),
        pltpu.VMEM((VROWS, ROWW), jnp.float32),
        pltpu.VMEM((SB, ROWW), jnp.float32),
        pltpu.VMEM((VROWS, ROWW), jnp.float32),
        pltpu.VMEM((NEG, L), jnp.float32),
        pltpu.VMEM((SCORES_PER_CHUNK,), jnp.float32),
        pltpu.SemaphoreType.DMA,
        pltpu.SemaphoreType.DMA,
    ],
)
def _sc_scores(pos_u_h, vall_h, uw_h, vw_h, out_h,
               idx_u, idx_v,
               bu0, bv0, bu1, bv1,
               accb, scores, sem0, sem1):
    _sc_scores_kernel(pos_u_h, vall_h, uw_h, vw_h, out_h,
                      idx_u, idx_v,
                      bu0, bv0, bu1, bv1,
                      accb, scores, sem0, sem1)


def _tc_loss_body(x_ref, o_ref):
    x = x_ref[...]
    ls = jnp.minimum(x, 0.0) - jnp.log1p(jnp.exp(-jnp.abs(x)))
    o_ref[...] = (-jnp.sum(ls)).reshape(1, 1)


def kernel(pos_u, pos_v, neg_v, u_weight, v_weight):
    pos_u = pos_u.astype(jnp.int32)
    pos_v = pos_v.astype(jnp.int32)
    neg_v = neg_v.astype(jnp.int32)
    pv = pos_v.reshape(NW, NCHUNK, L)
    ng = neg_v.reshape(NW, NCHUNK, L * NEG)
    vall = jnp.concatenate([pv, ng], axis=2).reshape(-1)
    u2 = _tc_relayout(u_weight)
    v2 = _tc_relayout(v_weight)
    scores = _sc_scores(pos_u, vall, u2, v2)
    scores2d = scores.reshape(TOT_SCORES // 128, 128)
    loss = pl.pallas_call(
        _tc_loss_body,
        out_shape=jax.ShapeDtypeStruct((1, 1), jnp.float32),
    )(scores2d)
    return loss[0, 0]
